# baseline (speedup 1.0000x reference)
import functools

import jax
import jax.numpy as jnp
from jax import lax
from jax.experimental import pallas as pl
from jax.experimental.pallas import tpu as pltpu
from jax.experimental.pallas import tpu_sc as plsc

H = 256
BN = 1000


GC = 128


def _sc_segsum_call(n_dst, e_pad, h, ones_mode, x_src, srcp, dstp):
    n_pad = -(-(n_dst + 8) // 128) * 128
    zr = n_pad // 16
    epw = e_pad // 32
    nchunk = epw // GC

    def body(src_h, dst_h, xsrc_h, zeros_h, out_h, sbuf, dbuf, stage, sem):
        c = lax.axis_index("c")
        s = lax.axis_index("s")
        g = c * 16 + s

        if ones_mode:
            ones16 = jnp.ones((16,), jnp.float32)
            for r in range(GC):
                for cc in range(h // 16):
                    stage[r, pl.ds(cc * 16, 16)] = ones16

        pltpu.sync_copy(zeros_h, out_h.at[c, pl.ds(s * zr, zr)])
        plsc.subcore_barrier()

        def chunk(k, _):
            off = g * epw + k * GC
            pltpu.sync_copy(dst_h.at[pl.ds(off, GC)], dbuf)
            if not ones_mode:
                pltpu.sync_copy(src_h.at[pl.ds(off, GC)], sbuf)
                pltpu.async_copy(xsrc_h.at[sbuf], stage, sem).wait()
            for t in range(GC // 16):
                idx = dbuf[pl.ds(t * 16, 16)]
                pltpu.sync_copy(stage.at[pl.ds(t * 16, 16)],
                                out_h.at[c].at[idx], add=True)
            return 0

        lax.fori_loop(0, nchunk, chunk, 0)

    f = pl.kernel(
        body,
        out_type=jax.ShapeDtypeStruct((2, n_pad, h), jnp.float32),
        mesh=plsc.VectorSubcoreMesh(core_axis_name="c",
                                    subcore_axis_name="s"),
        compiler_params=pltpu.CompilerParams(needs_layout_passes=False),
        scratch_types=[
            pltpu.VMEM((GC,), jnp.int32),
            pltpu.VMEM((GC,), jnp.int32),
            pltpu.VMEM((GC, h), jnp.float32),
            pltpu.SemaphoreType.DMA,
        ],
    )
    zeros = jnp.zeros((zr, h), jnp.float32)
    return f(srcp, dstp, x_src, zeros)


def _pad_edges(ei, n_dst):
    e = ei.shape[1]
    e_pad = -(-e // 4096) * 4096
    src = jnp.pad(ei[0], (0, e_pad - e)).astype(jnp.int32)
    dst = jnp.pad(ei[1], (0, e_pad - e),
                  constant_values=n_dst).astype(jnp.int32)
    return src, dst, e_pad


def _sc_counts(dstp, e_pad, n_dst):
    out = _sc_segsum_call(n_dst, e_pad, H, True,
                          jnp.zeros((8, H), jnp.float32), dstp, dstp)
    return out[:, :n_dst, 0]


def _sc_segsum(x_src, srcp, dstp, e_pad, n_dst):
    out = _sc_segsum_call(n_dst, e_pad, H, False, x_src, srcp, dstp)
    return out[:, :n_dst]


def _embed_body(p1_ref, p2_ref, p3_ref, t_ref, o_ref):
    i1 = p1_ref[0, 0, :]
    i2 = p2_ref[0, 0, :]
    i3 = p3_ref[0, 0, :]
    cols = lax.broadcasted_iota(jnp.int32, (BN, 128), 1)
    oh = ((cols == i1[:, None]).astype(jnp.float32)
          + (cols == i2[:, None]).astype(jnp.float32)
          + (cols == i3[:, None]).astype(jnp.float32))
    o_ref[...] = jnp.dot(oh, t_ref[...], preferred_element_type=jnp.float32)


def _embed(p1, p2, p3, table_pad):
    n = p1.shape[0]
    nb = n // BN
    p1 = p1.reshape(nb, 1, BN)
    p2 = p2.reshape(nb, 1, BN)
    p3 = p3.reshape(nb, 1, BN)
    return pl.pallas_call(
        _embed_body,
        grid=(nb,),
        in_specs=[
            pl.BlockSpec((1, 1, BN), lambda i: (i, 0, 0)),
            pl.BlockSpec((1, 1, BN), lambda i: (i, 0, 0)),
            pl.BlockSpec((1, 1, BN), lambda i: (i, 0, 0)),
            pl.BlockSpec((128, H), lambda i: (0, 0)),
        ],
        out_specs=pl.BlockSpec((BN, H), lambda i: (i, 0)),
        out_shape=jax.ShapeDtypeStruct((n, H), jnp.float32),
    )(p1, p2, p3, table_pad)


def _update_body(r, s_ref, cnt_ref, x_ref, wl_ref, wr_ref, b_ref, o_ref):
    acc = jnp.dot(x_ref[...], wr_ref[...], preferred_element_type=jnp.float32)
    for j in range(r):
        cnt = cnt_ref[j, 0, 0, 0] + cnt_ref[j, 1, 0, 0]
        inv = 1.0 / jnp.maximum(cnt, 1.0)
        srow = (s_ref[j, 0] + s_ref[j, 1]) * inv[:, None]
        acc = acc + jnp.dot(srow, wl_ref[j], preferred_element_type=jnp.float32)
    o_ref[...] = jnp.maximum(acc + b_ref[0], 0.0)


def _update(s_stack, cnt_stack, x_dst, wl_stack, wr_sum, b_sum):
    r, _, n, _ = s_stack.shape
    nb = n // BN
    cnt4 = cnt_stack.reshape(r, 2, nb, 1, BN)
    return pl.pallas_call(
        functools.partial(_update_body, r),
        grid=(nb,),
        in_specs=[
            pl.BlockSpec((r, 2, BN, H), lambda i: (0, 0, i, 0)),
            pl.BlockSpec((r, 2, 1, 1, BN), lambda i: (0, 0, i, 0, 0)),
            pl.BlockSpec((BN, H), lambda i: (i, 0)),
            pl.BlockSpec((r, H, H), lambda i: (0, 0, 0)),
            pl.BlockSpec((H, H), lambda i: (0, 0)),
            pl.BlockSpec((1, H), lambda i: (0, 0)),
        ],
        out_specs=pl.BlockSpec((BN, H), lambda i: (i, 0)),
        out_shape=jax.ShapeDtypeStruct((n, H), jnp.float32),
    )(s_stack, cnt4, x_dst, wl_stack, wr_sum, b_sum)


def _mlp_body(g_ref, w1_ref, b1_ref, w2_ref, b2_ref, w3_ref, b3_ref, o_ref):
    h = jnp.maximum(jnp.dot(g_ref[...], w1_ref[...],
                            preferred_element_type=jnp.float32) + b1_ref[0], 0.0)
    h = jnp.maximum(jnp.dot(h, w2_ref[...],
                            preferred_element_type=jnp.float32) + b2_ref[0], 0.0)
    o_ref[...] = jnp.dot(h, w3_ref[...],
                         preferred_element_type=jnp.float32) + b3_ref[0]


def _mlp(g, c1w, c1b, c2w, c2b, c3w, c3b):
    return pl.pallas_call(
        _mlp_body,
        out_shape=jax.ShapeDtypeStruct((g.shape[0], 10), jnp.float32),
    )(g, c1w, c1b.reshape(1, -1), c2w, c2b.reshape(1, -1),
      c3w, c3b.reshape(1, -1))


def _segsum(x, seg, n):
    return jax.ops.segment_sum(x, seg, num_segments=n)


def kernel(x_component, x_pin, x_subcircuit, x_net, ei_cp, ei_pc, ei_sp,
           ei_ps, ei_pn, ei_np, batch, nte, cte, pte, Wl, bl, Wr,
           C1w, C1b, C2w, C2b, C3w, C3b):
    NC = x_component.shape[0]
    NP = x_pin.shape[0]
    NS = x_subcircuit.shape[0]
    NN = x_net.shape[0]
    G = 64

    table = jnp.concatenate([nte, cte, pte], axis=0)
    table_pad = jnp.zeros((128, H), jnp.float32).at[:26].set(table)
    xs = jnp.concatenate([x_component, x_pin, x_subcircuit, x_net], axis=0)
    p1 = xs[:, 0]
    ct = jnp.clip(xs[:, 1], 0)
    ct = ct.at[:NC].set(0)
    p2 = 4 + ct
    p3 = 13 + jnp.clip(xs[:, 2], 0)
    emb = _embed(p1.astype(jnp.int32), p2.astype(jnp.int32),
                 p3.astype(jnp.int32), table_pad)
    comp = emb[:NC]
    pin = emb[NC:NC + NP]
    sub = emb[NC + NP:NC + NP + NS]
    net = emb[NC + NP + NS:]

    sp_cp, dp_cp, ep_cp = _pad_edges(ei_cp, NP)
    sp_pc, dp_pc, ep_pc = _pad_edges(ei_pc, NC)
    sp_sp, dp_sp, ep_sp = _pad_edges(ei_sp, NP)
    sp_ps, dp_ps, ep_ps = _pad_edges(ei_ps, NS)
    sp_pn, dp_pn, ep_pn = _pad_edges(ei_pn, NN)
    sp_np, dp_np, ep_np = _pad_edges(ei_np, NP)

    c_cp = _sc_counts(dp_cp, ep_cp, NP)
    c_sp = _sc_counts(dp_sp, ep_sp, NP)
    c_np = _sc_counts(dp_np, ep_np, NP)
    c_pc = _sc_counts(dp_pc, ep_pc, NC)
    c_ps = _sc_counts(dp_ps, ep_ps, NS)
    c_pn = _sc_counts(dp_pn, ep_pn, NN)
    cnt_pin = jnp.stack([c_cp, c_sp, c_np])

    for i in range(3):
        s_cp = _sc_segsum(comp, sp_cp, dp_cp, ep_cp, NP)
        s_sp = _sc_segsum(sub, sp_sp, dp_sp, ep_sp, NP)
        s_np = _sc_segsum(net, sp_np, dp_np, ep_np, NP)
        s_pc = _sc_segsum(pin, sp_pc, dp_pc, ep_pc, NC)
        s_ps = _sc_segsum(pin, sp_ps, dp_ps, ep_ps, NS)
        s_pn = _sc_segsum(pin, sp_pn, dp_pn, ep_pn, NN)

        pin_new = _update(
            jnp.stack([s_cp, s_sp, s_np]), cnt_pin, pin,
            jnp.stack([Wl[i, 0], Wl[i, 2], Wl[i, 5]]),
            Wr[i, 0] + Wr[i, 2] + Wr[i, 5],
            (bl[i, 0] + bl[i, 2] + bl[i, 5]).reshape(1, H))
        comp_new = _update(s_pc[None], c_pc[None], comp, Wl[i, 1][None],
                           Wr[i, 1], bl[i, 1].reshape(1, H))
        sub_new = _update(s_ps[None], c_ps[None], sub, Wl[i, 3][None],
                          Wr[i, 3], bl[i, 3].reshape(1, H))
        net_new = _update(s_pn[None], c_pn[None], net, Wl[i, 4][None],
                          Wr[i, 4], bl[i, 4].reshape(1, H))
        comp, pin, sub, net = comp_new, pin_new, sub_new, net_new

    s = _segsum(comp, batch, G)
    cnt = _segsum(jnp.ones((NC,), jnp.float32), batch, G)
    mean_pool = s / jnp.maximum(cnt, 1.0)[:, None]
    mx = jax.ops.segment_max(comp, batch, num_segments=G)
    max_pool = jnp.where(jnp.isfinite(mx), mx, 0.0)
    g = jnp.concatenate([mean_pool, max_pool], axis=1)
    return _mlp(g, C1w, C1b, C2w, C2b, C3w, C3b)


# speedup vs baseline: 7.5372x; 7.5372x over previous
import functools

import jax
import jax.numpy as jnp
from jax import lax
from jax.experimental import pallas as pl
from jax.experimental.pallas import tpu as pltpu
from jax.experimental.pallas import tpu_sc as plsc

H = 256
BN = 1000


GC = 128


def _sc_segsum_call(n_dst, e_pad, x_src, srcp, dstp):
    h = H
    n_pad = -(-(n_dst + 8) // 2048) * 2048
    zr = n_pad // 16
    epw = e_pad // 32
    nchunk = epw // GC

    def body(src_h, dst_h, xsrc_h, out_h, sall, dall, st0, st1, zbuf,
             gsem0, gsem1, asem):
        c = lax.axis_index("c")
        s = lax.axis_index("s")
        g = c * 16 + s

        zero16 = jnp.zeros((16,), jnp.float32)
        for r in range(GC):
            for cc in range(h // 16):
                zbuf[r, pl.ds(cc * 16, 16)] = zero16
        zoff = pl.multiple_of(s * zr, 8)
        for r in range(zr // GC):
            pltpu.sync_copy(zbuf, out_h.at[c, pl.ds(zoff + r * GC, GC)])
        eoff = pl.multiple_of(g * epw, 128)
        pltpu.sync_copy(src_h.at[pl.ds(eoff, epw)], sall)
        pltpu.sync_copy(dst_h.at[pl.ds(eoff, epw)], dall)
        plsc.subcore_barrier()

        def gather(k, buf, sem):
            koff = pl.multiple_of(k * GC, GC)
            return pltpu.make_async_copy(
                xsrc_h.at[sall.at[pl.ds(koff, GC)]], buf, sem)

        def adds(k, buf):
            ds = []
            for t in range(GC // 16):
                idx = dall[pl.ds(k * GC + t * 16, 16)]
                ds.append(pltpu.make_async_copy(
                    buf.at[pl.ds(t * 16, 16)], out_h.at[c].at[idx], asem))
            return ds

        gather(0, st0, gsem0).start()

        def chunk2(k2, _):
            k = k2 * 2

            @pl.when(k > 0)
            def _():
                for d in adds(k - 1, st1):
                    d.wait()

            @pl.when(k + 1 < nchunk)
            def _():
                gather(k + 1, st1, gsem1).start()

            gather(k, st0, gsem0).wait()
            for d in adds(k, st0):
                d.start()
            for d in adds(k, st0):
                d.wait()

            @pl.when(k + 2 < nchunk)
            def _():
                gather(k + 2, st0, gsem0).start()

            gather(k + 1, st1, gsem1).wait()
            for d in adds(k + 1, st1):
                d.start()
            return 0

        lax.fori_loop(0, nchunk // 2, chunk2, 0)
        for d in adds(nchunk - 1, st1):
            d.wait()

    f = pl.kernel(
        body,
        out_type=jax.ShapeDtypeStruct((2, n_pad, h), jnp.float32),
        mesh=plsc.VectorSubcoreMesh(core_axis_name="c",
                                    subcore_axis_name="s"),
        compiler_params=pltpu.CompilerParams(needs_layout_passes=False),
        scratch_types=[
            pltpu.VMEM((epw,), jnp.int32),
            pltpu.VMEM((epw,), jnp.int32),
            pltpu.VMEM((GC, h), jnp.float32),
            pltpu.VMEM((GC, h), jnp.float32),
            pltpu.VMEM((GC, h), jnp.float32),
            pltpu.SemaphoreType.DMA,
            pltpu.SemaphoreType.DMA,
            pltpu.SemaphoreType.DMA,
        ],
    )
    return f(srcp, dstp, x_src)


def _pad_edges(ei, n_dst):
    e = ei.shape[1]
    e_pad = -(-e // 8192) * 8192
    src = jnp.pad(ei[0], (0, e_pad - e)).astype(jnp.int32)
    dst = jnp.pad(ei[1], (0, e_pad - e),
                  constant_values=n_dst).astype(jnp.int32)
    return src, dst, e_pad


def _sc_counts_all(rels):
    pads = [-(-(n + 8) // 2048) * 2048 for _, _, n in rels]
    max_pad = max(pads)
    max_epw = max(e // 32 for _, e, _ in rels)

    def body(*refs):
        nrel = len(rels)
        dst_hs = refs[:nrel]
        outs = refs[nrel:2 * nrel]
        dall, hist, tmp, spm = refs[2 * nrel:2 * nrel + 4]
        c = lax.axis_index("c")
        s = lax.axis_index("s")
        g = c * 16 + s
        lanes = lax.iota(jnp.int32, 16)
        zero16 = jnp.zeros((16,), jnp.float32)
        one16 = jnp.ones((16,), jnp.float32)
        masks = [lanes == j for j in range(16)]

        for ri, (dstp_h, out_h, (_, e_pad, n_dst)) in enumerate(
                zip(dst_hs, outs, rels)):
            n_pad = pads[ri]
            epw = e_pad // 32
            sw = n_pad // 16

            def zb(i, _):
                hist[pl.ds(i * 16, 16)] = zero16
                return 0

            lax.fori_loop(0, n_pad // 16, zb, 0)
            pltpu.sync_copy(dstp_h.at[pl.ds(pl.multiple_of(g * epw, 128),
                                             epw)],
                            dall.at[pl.ds(0, epw)])

            def hgrp(gi, _):
                d = dall[pl.ds(gi * 16, 16)]
                for j in range(16):
                    plsc.addupdate_scatter(hist, [d], one16, mask=masks[j])
                return 0

            lax.fori_loop(0, epw // 16, hgrp, 0)
            pltpu.sync_copy(hist.at[pl.ds(0, n_pad)],
                            spm.at[pl.ds(pl.multiple_of(s * n_pad, 128),
                                         n_pad)])
            plsc.subcore_barrier()

            soff = pl.multiple_of(s * sw, 8)
            pltpu.sync_copy(spm.at[pl.ds(soff, sw)], tmp.at[pl.ds(0, sw)])
            for t2 in range(1, 16):
                pltpu.sync_copy(spm.at[pl.ds(t2 * n_pad + soff, sw)],
                                tmp.at[pl.ds(sw, sw)])

                def red(i, _):
                    a = tmp[pl.ds(i * 16, 16)]
                    b = tmp[pl.ds(sw + i * 16, 16)]
                    tmp[pl.ds(i * 16, 16)] = a + b
                    return 0

                lax.fori_loop(0, sw // 16, red, 0)
            woff = pl.multiple_of(c * n_pad + soff, 8)
            pltpu.sync_copy(tmp.at[pl.ds(0, sw)], out_h.at[pl.ds(woff, sw)])
            plsc.subcore_barrier()

    f = pl.kernel(
        body,
        out_type=[jax.ShapeDtypeStruct((2 * p,), jnp.float32)
                  for p in pads],
        mesh=plsc.VectorSubcoreMesh(core_axis_name="c",
                                    subcore_axis_name="s"),
        compiler_params=pltpu.CompilerParams(needs_layout_passes=False),
        scratch_types=[
            pltpu.VMEM((max_epw,), jnp.int32),
            pltpu.VMEM((max_pad,), jnp.float32),
            pltpu.VMEM((2 * (max_pad // 16),), jnp.float32),
            pltpu.VMEM_SHARED((16 * max_pad,), jnp.float32),
        ],
    )
    outs = f(*[d for d, _, _ in rels])
    return [o.reshape(2, p)[:, :n]
            for o, p, (_, _, n) in zip(outs, pads, rels)]


def _sc_segsum(x_src, srcp, dstp, e_pad, n_dst):
    out = _sc_segsum_call(n_dst, e_pad, x_src, srcp, dstp)
    return out[:, :n_dst]


def _embed_body(p1_ref, p2_ref, p3_ref, t_ref, o_ref):
    i1 = p1_ref[0, 0, :]
    i2 = p2_ref[0, 0, :]
    i3 = p3_ref[0, 0, :]
    cols = lax.broadcasted_iota(jnp.int32, (BN, 128), 1)
    oh = ((cols == i1[:, None]).astype(jnp.float32)
          + (cols == i2[:, None]).astype(jnp.float32)
          + (cols == i3[:, None]).astype(jnp.float32))
    o_ref[...] = jnp.dot(oh, t_ref[...], preferred_element_type=jnp.float32)


def _embed(p1, p2, p3, table_pad):
    n = p1.shape[0]
    nb = n // BN
    p1 = p1.reshape(nb, 1, BN)
    p2 = p2.reshape(nb, 1, BN)
    p3 = p3.reshape(nb, 1, BN)
    return pl.pallas_call(
        _embed_body,
        grid=(nb,),
        in_specs=[
            pl.BlockSpec((1, 1, BN), lambda i: (i, 0, 0)),
            pl.BlockSpec((1, 1, BN), lambda i: (i, 0, 0)),
            pl.BlockSpec((1, 1, BN), lambda i: (i, 0, 0)),
            pl.BlockSpec((128, H), lambda i: (0, 0)),
        ],
        out_specs=pl.BlockSpec((BN, H), lambda i: (i, 0)),
        out_shape=jax.ShapeDtypeStruct((n, H), jnp.float32),
    )(p1, p2, p3, table_pad)


def _update_body(r, s_ref, cnt_ref, x_ref, wl_ref, wr_ref, b_ref, o_ref):
    acc = jnp.dot(x_ref[...], wr_ref[...], preferred_element_type=jnp.float32)
    for j in range(r):
        cnt = cnt_ref[j, 0, 0, 0] + cnt_ref[j, 1, 0, 0]
        inv = 1.0 / jnp.maximum(cnt, 1.0)
        srow = (s_ref[j, 0] + s_ref[j, 1]) * inv[:, None]
        acc = acc + jnp.dot(srow, wl_ref[j], preferred_element_type=jnp.float32)
    o_ref[...] = jnp.maximum(acc + b_ref[0], 0.0)


def _update(s_stack, cnt_stack, x_dst, wl_stack, wr_sum, b_sum):
    r, _, n, _ = s_stack.shape
    nb = n // BN
    cnt4 = cnt_stack.reshape(r, 2, nb, 1, BN)
    return pl.pallas_call(
        functools.partial(_update_body, r),
        grid=(nb,),
        in_specs=[
            pl.BlockSpec((r, 2, BN, H), lambda i: (0, 0, i, 0)),
            pl.BlockSpec((r, 2, 1, 1, BN), lambda i: (0, 0, i, 0, 0)),
            pl.BlockSpec((BN, H), lambda i: (i, 0)),
            pl.BlockSpec((r, H, H), lambda i: (0, 0, 0)),
            pl.BlockSpec((H, H), lambda i: (0, 0)),
            pl.BlockSpec((1, H), lambda i: (0, 0)),
        ],
        out_specs=pl.BlockSpec((BN, H), lambda i: (i, 0)),
        out_shape=jax.ShapeDtypeStruct((n, H), jnp.float32),
    )(s_stack, cnt4, x_dst, wl_stack, wr_sum, b_sum)


def _mlp_body(g_ref, w1_ref, b1_ref, w2_ref, b2_ref, w3_ref, b3_ref, o_ref):
    h = jnp.maximum(jnp.dot(g_ref[...], w1_ref[...],
                            preferred_element_type=jnp.float32) + b1_ref[0], 0.0)
    h = jnp.maximum(jnp.dot(h, w2_ref[...],
                            preferred_element_type=jnp.float32) + b2_ref[0], 0.0)
    o_ref[...] = jnp.dot(h, w3_ref[...],
                         preferred_element_type=jnp.float32) + b3_ref[0]


def _mlp(g, c1w, c1b, c2w, c2b, c3w, c3b):
    return pl.pallas_call(
        _mlp_body,
        out_shape=jax.ShapeDtypeStruct((g.shape[0], 10), jnp.float32),
    )(g, c1w, c1b.reshape(1, -1), c2w, c2b.reshape(1, -1),
      c3w, c3b.reshape(1, -1))


def _segsum(x, seg, n):
    return jax.ops.segment_sum(x, seg, num_segments=n)


def kernel(x_component, x_pin, x_subcircuit, x_net, ei_cp, ei_pc, ei_sp,
           ei_ps, ei_pn, ei_np, batch, nte, cte, pte, Wl, bl, Wr,
           C1w, C1b, C2w, C2b, C3w, C3b):
    NC = x_component.shape[0]
    NP = x_pin.shape[0]
    NS = x_subcircuit.shape[0]
    NN = x_net.shape[0]
    G = 64

    table = jnp.concatenate([nte, cte, pte], axis=0)
    table_pad = jnp.zeros((128, H), jnp.float32).at[:26].set(table)
    xs = jnp.concatenate([x_component, x_pin, x_subcircuit, x_net], axis=0)
    p1 = xs[:, 0]
    ct = jnp.clip(xs[:, 1], 0)
    ct = ct.at[:NC].set(0)
    p2 = 4 + ct
    p3 = 13 + jnp.clip(xs[:, 2], 0)
    emb = _embed(p1.astype(jnp.int32), p2.astype(jnp.int32),
                 p3.astype(jnp.int32), table_pad)
    comp = emb[:NC]
    pin = emb[NC:NC + NP]
    sub = emb[NC + NP:NC + NP + NS]
    net = emb[NC + NP + NS:]

    sp_cp, dp_cp, ep_cp = _pad_edges(ei_cp, NP)
    sp_pc, dp_pc, ep_pc = _pad_edges(ei_pc, NC)
    sp_sp, dp_sp, ep_sp = _pad_edges(ei_sp, NP)
    sp_ps, dp_ps, ep_ps = _pad_edges(ei_ps, NS)
    sp_pn, dp_pn, ep_pn = _pad_edges(ei_pn, NN)
    sp_np, dp_np, ep_np = _pad_edges(ei_np, NP)

    c_cp, c_sp, c_np, c_pc, c_ps, c_pn = _sc_counts_all([
        (dp_cp, ep_cp, NP), (dp_sp, ep_sp, NP), (dp_np, ep_np, NP),
        (dp_pc, ep_pc, NC), (dp_ps, ep_ps, NS), (dp_pn, ep_pn, NN)])
    cnt_pin = jnp.stack([c_cp, c_sp, c_np])

    for i in range(3):
        s_cp = _sc_segsum(comp, sp_cp, dp_cp, ep_cp, NP)
        s_sp = _sc_segsum(sub, sp_sp, dp_sp, ep_sp, NP)
        s_np = _sc_segsum(net, sp_np, dp_np, ep_np, NP)
        s_pc = _sc_segsum(pin, sp_pc, dp_pc, ep_pc, NC)
        s_ps = _sc_segsum(pin, sp_ps, dp_ps, ep_ps, NS)
        s_pn = _sc_segsum(pin, sp_pn, dp_pn, ep_pn, NN)

        pin_new = _update(
            jnp.stack([s_cp, s_sp, s_np]), cnt_pin, pin,
            jnp.stack([Wl[i, 0], Wl[i, 2], Wl[i, 5]]),
            Wr[i, 0] + Wr[i, 2] + Wr[i, 5],
            (bl[i, 0] + bl[i, 2] + bl[i, 5]).reshape(1, H))
        comp_new = _update(s_pc[None], c_pc[None], comp, Wl[i, 1][None],
                           Wr[i, 1], bl[i, 1].reshape(1, H))
        sub_new = _update(s_ps[None], c_ps[None], sub, Wl[i, 3][None],
                          Wr[i, 3], bl[i, 3].reshape(1, H))
        net_new = _update(s_pn[None], c_pn[None], net, Wl[i, 4][None],
                          Wr[i, 4], bl[i, 4].reshape(1, H))
        comp, pin, sub, net = comp_new, pin_new, sub_new, net_new

    s = _segsum(comp, batch, G)
    cnt = _segsum(jnp.ones((NC,), jnp.float32), batch, G)
    mean_pool = s / jnp.maximum(cnt, 1.0)[:, None]
    mx = jax.ops.segment_max(comp, batch, num_segments=G)
    max_pool = jnp.where(jnp.isfinite(mx), mx, 0.0)
    g = jnp.concatenate([mean_pool, max_pool], axis=1)
    return _mlp(g, C1w, C1b, C2w, C2b, C3w, C3b)
